# baseline (speedup 1.0000x reference)
import math

import jax
import jax.numpy as jnp
from jax import lax
from jax.experimental import pallas as pl
from jax.experimental.pallas import tpu as pltpu
import jax.experimental.pallas.tpu_sc as plsc

_S = 64.0
_M2 = 0.5
_IGNORE = 98

_BN = 2048
_L = 16
_NW = 32
_GPW = 2


def _sc_gather_body(logits_hbm, labels_hbm, vals_hbm, lab_v, win_v, val_v, sem):
    c = lax.axis_index("c")
    s = lax.axis_index("s")
    wid = s * 2 + c
    row0 = wid * (_GPW * _L)
    pltpu.sync_copy(labels_hbm.at[pl.ds(row0, _GPW * _L)], lab_v)
    lanes = lax.iota(jnp.int32, _L)
    for g in range(_GPW):
        labs = lab_v[pl.ds(g * _L, _L)]
        copies = []
        for k in range(_L):
            lab_k = labs[k]
            ct_k = (lab_k >> 7) << 7
            rt_k = row0 + g * _L + (k & ~7)
            copies.append(
                pltpu.async_copy(
                    logits_hbm.at[
                        pl.ds(pl.multiple_of(rt_k, 8), 8),
                        pl.ds(pl.multiple_of(ct_k, 128), 128),
                    ],
                    win_v.at[k],
                    sem,
                )
            )
        for cp in copies:
            cp.wait()
        acc = jnp.zeros((_L,), jnp.float32)
        for k in range(_L):
            lab_k = labs[k]
            seg_k = ((lab_k & 127) >> 4) << 4
            row16 = win_v[k, k & 7, pl.ds(pl.multiple_of(seg_k, 16), _L)]
            lane_k = jnp.full((_L,), lab_k & 15, jnp.int32)
            v_b = lax.gather(
                row16,
                lane_k[:, None],
                lax.GatherDimensionNumbers(
                    offset_dims=(),
                    collapsed_slice_dims=(0,),
                    start_index_map=(0,),
                ),
                slice_sizes=(1,),
                mode=lax.GatherScatterMode.PROMISE_IN_BOUNDS,
            )
            acc = jnp.where(lanes == k, v_b, acc)
        val_v[pl.ds(g * _L, _L)] = acc
    pltpu.sync_copy(val_v, vals_hbm.at[pl.ds(row0, _GPW * _L)])


def _sc_gather(logits, labels):
    mesh = plsc.VectorSubcoreMesh(core_axis_name="c", subcore_axis_name="s")
    import functools
    @functools.partial(
        pl.kernel,
        out_type=jax.ShapeDtypeStruct((labels.shape[0],), jnp.float32),
        mesh=mesh,
        scratch_types=[
            pltpu.VMEM((_GPW * _L,), jnp.int32),
            pltpu.VMEM((_L, 8, 128), jnp.float32),
            pltpu.VMEM((_GPW * _L,), jnp.float32),
            pltpu.SemaphoreType.DMA,
        ],
    )
    def run(logits_hbm, labels_hbm, vals_hbm, lab_v, win_v, val_v, sem):
        _sc_gather_body(logits_hbm, labels_hbm, vals_hbm, lab_v, win_v, val_v, sem)
    return run(logits, labels)


def _tc_body(labels_ref, vals_ref, x_ref, o_ref):
    j = pl.program_id(0)
    x = x_ref[...]
    lab = labels_ref[...]
    g = vals_ref[...]
    cols = jax.lax.broadcasted_iota(jnp.int32, x.shape, 1) + j * x.shape[1]
    hit = lab == cols
    cm, sm = math.cos(_M2), math.sin(_M2)
    adj = g * jnp.float32(cm) - jnp.sqrt(jnp.maximum(1.0 - g * g, 0.0)) * jnp.float32(sm)
    fixed = jnp.where(lab != _IGNORE, adj, g)
    o_ref[...] = jnp.float32(_S) * jnp.where(hit, fixed, x)


@jax.jit
def kernel(logits, labels):
    B, V = logits.shape
    labels_i32 = labels.astype(jnp.int32)
    vals = _sc_gather(logits, labels_i32)
    labels2d = labels_i32.reshape(B, 1)
    vals2d = vals.reshape(B, 1)
    return pl.pallas_call(
        _tc_body,
        grid=(pl.cdiv(V, _BN),),
        in_specs=[
            pl.BlockSpec((B, 1), lambda j: (0, 0)),
            pl.BlockSpec((B, 1), lambda j: (0, 0)),
            pl.BlockSpec((B, _BN), lambda j: (0, j)),
        ],
        out_specs=pl.BlockSpec((B, _BN), lambda j: (0, j)),
        out_shape=jax.ShapeDtypeStruct((B, V), jnp.float32),
    )(labels2d, vals2d, logits)


# speedup vs baseline: 1.0021x; 1.0021x over previous
import math

import jax
import jax.numpy as jnp
from jax import lax
from jax.experimental import pallas as pl
from jax.experimental.pallas import tpu as pltpu
import jax.experimental.pallas.tpu_sc as plsc

_S = 64.0
_M2 = 0.5
_IGNORE = 98

_BN = 2048
_L = 16
_NW = 32
_GPW = 2


def _sc_gather_body(logits_hbm, labels_hbm, vals_hbm, lab_v, win_v, val_v, sem):
    c = lax.axis_index("c")
    s = lax.axis_index("s")
    wid = s * 2 + c
    row0 = wid * (_GPW * _L)
    pltpu.sync_copy(labels_hbm.at[pl.ds(row0, _GPW * _L)], lab_v)
    lanes = lax.iota(jnp.int32, _L)
    labs_g = [lab_v[pl.ds(g * _L, _L)] for g in range(_GPW)]
    copies = []
    for g in range(_GPW):
        for k in range(_L):
            lab_k = labs_g[g][k]
            ct_k = (lab_k >> 7) << 7
            rt_k = row0 + g * _L + (k & ~7)
            copies.append(
                pltpu.async_copy(
                    logits_hbm.at[
                        pl.ds(pl.multiple_of(rt_k, 8), 8),
                        pl.ds(pl.multiple_of(ct_k, 128), 128),
                    ],
                    win_v.at[g * _L + k],
                    sem,
                )
            )
    for cp in copies:
        cp.wait()
    for g in range(_GPW):
        acc = jnp.zeros((_L,), jnp.float32)
        for k in range(_L):
            lab_k = labs_g[g][k]
            seg_k = ((lab_k & 127) >> 4) << 4
            row16 = win_v[g * _L + k, k & 7, pl.ds(pl.multiple_of(seg_k, 16), _L)]
            lane_k = jnp.full((_L,), lab_k & 15, jnp.int32)
            v_b = lax.gather(
                row16,
                lane_k[:, None],
                lax.GatherDimensionNumbers(
                    offset_dims=(),
                    collapsed_slice_dims=(0,),
                    start_index_map=(0,),
                ),
                slice_sizes=(1,),
                mode=lax.GatherScatterMode.PROMISE_IN_BOUNDS,
            )
            acc = jnp.where(lanes == k, v_b, acc)
        val_v[pl.ds(g * _L, _L)] = acc
    pltpu.sync_copy(val_v, vals_hbm.at[pl.ds(row0, _GPW * _L)])


def _sc_gather(logits, labels):
    mesh = plsc.VectorSubcoreMesh(core_axis_name="c", subcore_axis_name="s")
    import functools
    @functools.partial(
        pl.kernel,
        out_type=jax.ShapeDtypeStruct((labels.shape[0],), jnp.float32),
        mesh=mesh,
        scratch_types=[
            pltpu.VMEM((_GPW * _L,), jnp.int32),
            pltpu.VMEM((_GPW * _L, 8, 128), jnp.float32),
            pltpu.VMEM((_GPW * _L,), jnp.float32),
            pltpu.SemaphoreType.DMA,
        ],
    )
    def run(logits_hbm, labels_hbm, vals_hbm, lab_v, win_v, val_v, sem):
        _sc_gather_body(logits_hbm, labels_hbm, vals_hbm, lab_v, win_v, val_v, sem)
    return run(logits, labels)


def _tc_body(labels_ref, vals_ref, x_ref, o_ref):
    j = pl.program_id(0)
    x = x_ref[...]
    lab = labels_ref[...]
    g = vals_ref[...]
    cols = jax.lax.broadcasted_iota(jnp.int32, x.shape, 1) + j * x.shape[1]
    hit = lab == cols
    cm, sm = math.cos(_M2), math.sin(_M2)
    adj = g * jnp.float32(cm) - jnp.sqrt(jnp.maximum(1.0 - g * g, 0.0)) * jnp.float32(sm)
    fixed = jnp.where(lab != _IGNORE, adj, g)
    o_ref[...] = jnp.float32(_S) * jnp.where(hit, fixed, x)


@jax.jit
def kernel(logits, labels):
    B, V = logits.shape
    labels_i32 = labels.astype(jnp.int32)
    vals = _sc_gather(logits, labels_i32)
    labels2d = labels_i32.reshape(B, 1)
    vals2d = vals.reshape(B, 1)
    return pl.pallas_call(
        _tc_body,
        grid=(pl.cdiv(V, _BN),),
        in_specs=[
            pl.BlockSpec((B, 1), lambda j: (0, 0)),
            pl.BlockSpec((B, 1), lambda j: (0, 0)),
            pl.BlockSpec((B, _BN), lambda j: (0, j)),
        ],
        out_specs=pl.BlockSpec((B, _BN), lambda j: (0, j)),
        out_shape=jax.ShapeDtypeStruct((B, V), jnp.float32),
    )(labels2d, vals2d, logits)


# speedup vs baseline: 1.0025x; 1.0004x over previous
import math

import jax
import jax.numpy as jnp
from jax import lax
from jax.experimental import pallas as pl
from jax.experimental.pallas import tpu as pltpu
import jax.experimental.pallas.tpu_sc as plsc

_S = 64.0
_M2 = 0.5
_IGNORE = 98

_BN = 3072
_L = 16
_NW = 32
_GPW = 2


def _sc_gather_body(logits_hbm, labels_hbm, vals_hbm, lab_v, win_v, val_v, sem):
    c = lax.axis_index("c")
    s = lax.axis_index("s")
    wid = s * 2 + c
    row0 = wid * (_GPW * _L)
    pltpu.sync_copy(labels_hbm.at[pl.ds(row0, _GPW * _L)], lab_v)
    lanes = lax.iota(jnp.int32, _L)
    labs_g = [lab_v[pl.ds(g * _L, _L)] for g in range(_GPW)]
    copies = []
    for g in range(_GPW):
        for k in range(_L):
            lab_k = labs_g[g][k]
            ct_k = (lab_k >> 7) << 7
            rt_k = row0 + g * _L + (k & ~7)
            copies.append(
                pltpu.async_copy(
                    logits_hbm.at[
                        pl.ds(pl.multiple_of(rt_k, 8), 8),
                        pl.ds(pl.multiple_of(ct_k, 128), 128),
                    ],
                    win_v.at[g * _L + k],
                    sem,
                )
            )
    for cp in copies:
        cp.wait()
    for g in range(_GPW):
        acc = jnp.zeros((_L,), jnp.float32)
        for k in range(_L):
            lab_k = labs_g[g][k]
            seg_k = ((lab_k & 127) >> 4) << 4
            row16 = win_v[g * _L + k, k & 7, pl.ds(pl.multiple_of(seg_k, 16), _L)]
            lane_k = jnp.full((_L,), lab_k & 15, jnp.int32)
            v_b = lax.gather(
                row16,
                lane_k[:, None],
                lax.GatherDimensionNumbers(
                    offset_dims=(),
                    collapsed_slice_dims=(0,),
                    start_index_map=(0,),
                ),
                slice_sizes=(1,),
                mode=lax.GatherScatterMode.PROMISE_IN_BOUNDS,
            )
            acc = jnp.where(lanes == k, v_b, acc)
        val_v[pl.ds(g * _L, _L)] = acc
    pltpu.sync_copy(val_v, vals_hbm.at[pl.ds(row0, _GPW * _L)])


def _sc_gather(logits, labels):
    mesh = plsc.VectorSubcoreMesh(core_axis_name="c", subcore_axis_name="s")
    import functools
    @functools.partial(
        pl.kernel,
        out_type=jax.ShapeDtypeStruct((labels.shape[0],), jnp.float32),
        mesh=mesh,
        scratch_types=[
            pltpu.VMEM((_GPW * _L,), jnp.int32),
            pltpu.VMEM((_GPW * _L, 8, 128), jnp.float32),
            pltpu.VMEM((_GPW * _L,), jnp.float32),
            pltpu.SemaphoreType.DMA,
        ],
    )
    def run(logits_hbm, labels_hbm, vals_hbm, lab_v, win_v, val_v, sem):
        _sc_gather_body(logits_hbm, labels_hbm, vals_hbm, lab_v, win_v, val_v, sem)
    return run(logits, labels)


def _tc_body(labels_ref, vals_ref, x_ref, o_ref):
    j = pl.program_id(0)
    x = x_ref[...]
    lab = labels_ref[...]
    g = vals_ref[...]
    cols = jax.lax.broadcasted_iota(jnp.int32, x.shape, 1) + j * x.shape[1]
    hit = lab == cols
    cm, sm = math.cos(_M2), math.sin(_M2)
    adj = g * jnp.float32(cm) - jnp.sqrt(jnp.maximum(1.0 - g * g, 0.0)) * jnp.float32(sm)
    fixed = jnp.where(lab != _IGNORE, adj, g)
    o_ref[...] = jnp.float32(_S) * jnp.where(hit, fixed, x)


@jax.jit
def kernel(logits, labels):
    B, V = logits.shape
    labels_i32 = labels.astype(jnp.int32)
    vals = _sc_gather(logits, labels_i32)
    labels2d = labels_i32.reshape(B, 1)
    vals2d = vals.reshape(B, 1)
    return pl.pallas_call(
        _tc_body,
        grid=(pl.cdiv(V, _BN),),
        in_specs=[
            pl.BlockSpec((B, 1), lambda j: (0, 0)),
            pl.BlockSpec((B, 1), lambda j: (0, 0)),
            pl.BlockSpec((B, _BN), lambda j: (0, j)),
        ],
        out_specs=pl.BlockSpec((B, _BN), lambda j: (0, j)),
        out_shape=jax.ShapeDtypeStruct((B, V), jnp.float32),
    )(labels2d, vals2d, logits)
